# Initial kernel scaffold; baseline (speedup 1.0000x reference)
import jax
import jax.numpy as jnp
from jax.experimental import pallas as pl


def kernel(x, Wg, bg, W1, b1, W2, b2):
    raise NotImplementedError("write your pallas kernel here")


# speedup vs baseline: 5.1908x; 5.1908x over previous
import functools
import math

import jax
import jax.numpy as jnp
from jax.experimental import pallas as pl
from jax.experimental.pallas import tpu as pltpu


def _first_occurrence_cummask(eq):
    e = eq.shape[-1]
    r = jax.lax.broadcasted_iota(jnp.int32, (e, e), 0)
    c = jax.lax.broadcasted_iota(jnp.int32, (e, e), 1)
    tri = (r <= c).astype(jnp.float32)
    return jax.lax.dot_general(
        eq.astype(jnp.float32), tri, (((1,), (0,)), ((), ())),
        precision=jax.lax.Precision.HIGHEST,
        preferred_element_type=jnp.float32)


def _f32_matmul(a, b):
    ah = a.astype(jnp.bfloat16)
    al = (a - ah.astype(jnp.float32)).astype(jnp.bfloat16)
    bh = b.astype(jnp.bfloat16)
    bl = (b - bh.astype(jnp.float32)).astype(jnp.bfloat16)
    dn = (((1,), (0,)), ((), ()))
    dot = functools.partial(jax.lax.dot_general, dimension_numbers=dn,
                            preferred_element_type=jnp.float32)
    return ((dot(al, bl) + dot(al, bh)) + dot(ah, bl)) + dot(ah, bh)


def _gelu_exact(h):
    return 0.5 * h * (1.0 + jax.lax.erf(h * (1.0 / math.sqrt(2.0))))


def _moe_body(x_ref, wg_ref, bg_ref, w1_ref, b1_ref, w2_ref, b2_ref,
              out_ref):
    num_experts = wg_ref.shape[-1]
    x = x_ref[...]
    logits = jax.lax.dot_general(
        x.astype(jnp.bfloat16), wg_ref[...].astype(jnp.bfloat16),
        (((1,), (0,)), ((), ())),
        preferred_element_type=jnp.float32) + bg_ref[...][None, :]
    m = jnp.max(logits, axis=-1, keepdims=True)
    ex = jnp.exp(logits - m)
    p = ex / jnp.sum(ex, axis=-1, keepdims=True)

    m1 = jnp.max(p, axis=-1, keepdims=True)
    eq1 = p == m1
    oh1 = eq1 & (_first_occurrence_cummask(eq1) == 1.0)
    pm = jnp.where(oh1, -jnp.inf, p)
    m2 = jnp.max(pm, axis=-1, keepdims=True)
    eq2 = pm == m2
    oh2 = eq2 & (_first_occurrence_cummask(eq2) == 1.0)
    denom = m1 + m2
    w8 = (jnp.where(oh1, m1, 0.0) + jnp.where(oh2, m2, 0.0)) / denom

    xb = x.astype(jnp.bfloat16)
    acc = jnp.zeros(out_ref.shape, jnp.float32)
    for e in range(num_experts):
        h = jax.lax.dot_general(
            xb, w1_ref[e], (((1,), (0,)), ((), ())),
            preferred_element_type=jnp.float32) + b1_ref[e][None, :]
        h = _gelu_exact(h).astype(jnp.bfloat16)
        o = jax.lax.dot_general(
            h, w2_ref[e], (((1,), (0,)), ((), ())),
            preferred_element_type=jnp.float32) + b2_ref[e][None, :]
        acc = acc + w8[:, e:e + 1] * o
    out_ref[...] = acc


def kernel(x, Wg, bg, W1, b1, W2, b2):
    n_tokens, d_in = x.shape
    num_experts = Wg.shape[-1]
    d_hid = W1.shape[-1]
    d_out = W2.shape[-1]
    bt = min(256, n_tokens)
    n_blocks = n_tokens // bt

    W1b = W1.astype(jnp.bfloat16)
    W2b = W2.astype(jnp.bfloat16)

    out = pl.pallas_call(
        _moe_body,
        grid=(n_blocks,),
        in_specs=[
            pl.BlockSpec((bt, d_in), lambda t: (t, 0)),
            pl.BlockSpec((d_in, num_experts), lambda t: (0, 0)),
            pl.BlockSpec((num_experts,), lambda t: (0,)),
            pl.BlockSpec((num_experts, d_in, d_hid), lambda t: (0, 0, 0)),
            pl.BlockSpec((num_experts, d_hid), lambda t: (0, 0)),
            pl.BlockSpec((num_experts, d_hid, d_out), lambda t: (0, 0, 0)),
            pl.BlockSpec((num_experts, d_out), lambda t: (0, 0)),
        ],
        out_specs=pl.BlockSpec((bt, d_out), lambda t: (t, 0)),
        out_shape=jax.ShapeDtypeStruct((n_tokens, d_out), jnp.float32),
        compiler_params=pltpu.CompilerParams(
            dimension_semantics=("arbitrary",)),
    )(x, Wg, bg, W1b, b1, W2b, b2)

    gate_probs = jax.nn.softmax(x @ Wg + bg, axis=-1)
    expert_usage = jnp.mean(gate_probs, axis=0)
    uniform = jnp.ones_like(expert_usage) / num_experts
    aux = jnp.sum(expert_usage * jnp.log(uniform)
                  - jnp.log(expert_usage) * uniform)
    return (out, aux)


# speedup vs baseline: 5.3631x; 1.0332x over previous
import dataclasses
import functools
import math

import jax
import jax.numpy as jnp
from jax import lax
from jax.experimental import pallas as pl
from jax.experimental.pallas import tpu as pltpu
from jax.experimental.pallas import tpu_sc as plsc

_NC, _NS, _LANES = 2, 16, 16
_NW = _NC * _NS


def _gelu_exact(h):
    return 0.5 * h * (1.0 + lax.erf(h * (1.0 / math.sqrt(2.0))))


def _first_occurrence_count(eq):
    e = eq.shape[-1]
    r = lax.broadcasted_iota(jnp.int32, (e, e), 0)
    c = lax.broadcasted_iota(jnp.int32, (e, e), 1)
    tri = (r <= c).astype(jnp.bfloat16)
    return lax.dot_general(eq.astype(jnp.bfloat16), tri,
                           (((1,), (0,)), ((), ())),
                           preferred_element_type=jnp.float32)


def _gate_body(x_ref, wg_ref, bg_ref,
               e0_ref, e1_ref, p0_ref, p1_ref, w0_ref, w1_ref,
               bex_ref, off_ref, cnt_ref, *, n_blocks, blk, nbmax):
    t = pl.program_id(0)
    bt = x_ref.shape[0]
    ne = wg_ref.shape[-1]

    @pl.when(t == 0)
    def _init():
        cnt_ref[...] = jnp.zeros_like(cnt_ref)

    x = x_ref[...]
    logits = lax.dot_general(
        x.astype(jnp.bfloat16), wg_ref[...].astype(jnp.bfloat16),
        (((1,), (0,)), ((), ())),
        preferred_element_type=jnp.float32) + bg_ref[...][None, :]
    m = jnp.max(logits, axis=-1, keepdims=True)
    ex = jnp.exp(logits - m)
    p = ex / jnp.sum(ex, axis=-1, keepdims=True)

    m1 = jnp.max(p, axis=-1, keepdims=True)
    eq1 = p == m1
    oh1 = eq1 & (_first_occurrence_count(eq1) == 1.0)
    pm = jnp.where(oh1, -jnp.inf, p)
    m2 = jnp.max(pm, axis=-1, keepdims=True)
    eq2 = pm == m2
    oh2 = eq2 & (_first_occurrence_count(eq2) == 1.0)
    oh1f = oh1.astype(jnp.float32)
    oh2f = oh2.astype(jnp.float32)
    denom = m1 + m2

    ei = lax.broadcasted_iota(jnp.int32, (1, ne), 1).astype(jnp.float32)
    e0 = jnp.sum(oh1f * ei, axis=-1)
    e1 = jnp.sum(oh2f * ei, axis=-1)

    r = lax.broadcasted_iota(jnp.int32, (bt, bt), 0)
    c = lax.broadcasted_iota(jnp.int32, (bt, bt), 1)
    ls = (r > c).astype(jnp.bfloat16)
    f01 = (oh1f + oh2f).astype(jnp.bfloat16)
    cbefore = lax.dot_general(ls, f01, (((1,), (0,)), ((), ())),
                              preferred_element_type=jnp.float32)
    base = cnt_ref[...]
    pos0 = jnp.sum(oh1f * (base + cbefore), axis=-1)
    pos1 = jnp.sum(oh2f * (base + cbefore + oh1f), axis=-1)
    cnt_ref[...] = base + jnp.sum(oh1f + oh2f, axis=0, keepdims=True)

    shape2 = e0_ref.shape
    e0_ref[...] = e0.astype(jnp.int32).reshape(shape2)
    e1_ref[...] = e1.astype(jnp.int32).reshape(shape2)
    p0_ref[...] = pos0.astype(jnp.int32).reshape(shape2)
    p1_ref[...] = pos1.astype(jnp.int32).reshape(shape2)
    w0_ref[...] = jnp.broadcast_to(m1 / denom, w0_ref.shape)
    w1_ref[...] = jnp.broadcast_to(m2 / denom, w1_ref.shape)

    @pl.when(t == n_blocks - 1)
    def _finish():
        cnt = cnt_ref[...]
        nb = jnp.floor((cnt + (blk - 1)) / blk)
        r8 = lax.broadcasted_iota(jnp.int32, (ne, ne), 0)
        c8 = lax.broadcasted_iota(jnp.int32, (ne, ne), 1)
        tri8 = (r8 < c8).astype(jnp.bfloat16)
        blkoff = lax.dot_general(nb.astype(jnp.bfloat16), tri8,
                                 (((1,), (0,)), ((), ())),
                                 preferred_element_type=jnp.float32)
        off = (blkoff * blk).astype(jnp.int32)
        off_ref[...] = jnp.concatenate(
            [off, jnp.zeros((1, 128 - ne), jnp.int32)], axis=1)
        li = lax.broadcasted_iota(jnp.int32, (1, 128), 1)
        acc = jnp.full((1, 128), -1, jnp.int32)
        for e in range(ne):
            acc = acc + (li >= blkoff[0, e].astype(jnp.int32)).astype(jnp.int32)
        bex_ref[...] = jnp.clip(acc, 0, ne - 1)


def _ffn_body(sp_ref, xg_ref, w1_ref, b1_ref, w2_ref, b2_ref, ws_ref,
              yo_ref):
    xb = xg_ref[...].astype(jnp.bfloat16)
    h = lax.dot_general(xb, w1_ref[0], (((1,), (0,)), ((), ())),
                        preferred_element_type=jnp.float32) + b1_ref[0]
    hb = _gelu_exact(h).astype(jnp.bfloat16)
    o = lax.dot_general(hb, w2_ref[0], (((1,), (0,)), ((), ())),
                        preferred_element_type=jnp.float32) + b2_ref[0]
    yo_ref[...] = o * ws_ref[...][:, 0:1]


def _scat_body(x_hbm, e0_hbm, e1_hbm, p0_hbm, p1_hbm, off_hbm,
               w0_hbm, w1_hbm,
               xg_hbm, ws_hbm, s0_hbm, s1_hbm,
               xbuf, w0buf, w1buf, e0b, e1b, p0b, p1b, s0b, s1b, offb,
               *, tpw, ch):
    wid = lax.axis_index("s") * _NC + lax.axis_index("c")
    base = wid * tpw
    pltpu.sync_copy(off_hbm.at[pl.ds(0, _LANES)], offb)

    @pl.loop(0, tpw, step=ch)
    def _chunk(c):
        t0 = base + c
        pltpu.sync_copy(x_hbm.at[pl.ds(t0, ch)], xbuf)
        pltpu.sync_copy(e0_hbm.at[pl.ds(t0, ch)], e0b)
        pltpu.sync_copy(e1_hbm.at[pl.ds(t0, ch)], e1b)
        pltpu.sync_copy(p0_hbm.at[pl.ds(t0, ch)], p0b)
        pltpu.sync_copy(p1_hbm.at[pl.ds(t0, ch)], p1b)
        pltpu.sync_copy(w0_hbm.at[pl.ds(t0, ch)], w0buf)
        pltpu.sync_copy(w1_hbm.at[pl.ds(t0, ch)], w1buf)
        for i in range(0, ch, _LANES):
            sl = pl.ds(i, _LANES)
            s0b[sl] = p0b[sl] + plsc.load_gather(offb, [e0b[sl]])
            s1b[sl] = p1b[sl] + plsc.load_gather(offb, [e1b[sl]])
        pltpu.sync_copy(xbuf, xg_hbm.at[s0b])
        pltpu.sync_copy(xbuf, xg_hbm.at[s1b])
        pltpu.sync_copy(w0buf, ws_hbm.at[s0b])
        pltpu.sync_copy(w1buf, ws_hbm.at[s1b])
        pltpu.sync_copy(s0b, s0_hbm.at[pl.ds(t0, ch)])
        pltpu.sync_copy(s1b, s1_hbm.at[pl.ds(t0, ch)])


def _gath_body(yo_hbm, s0_hbm, s1_hbm, out_hbm, y0b, y1b, s0b, s1b, sem,
               *, tpw, ch, d_out):
    wid = lax.axis_index("s") * _NC + lax.axis_index("c")
    base = wid * tpw

    @pl.loop(0, tpw, step=ch)
    def _chunk(c):
        t0 = base + c
        pltpu.sync_copy(s0_hbm.at[pl.ds(t0, ch)], s0b)
        pltpu.sync_copy(s1_hbm.at[pl.ds(t0, ch)], s1b)
        pltpu.async_copy(yo_hbm.at[s0b], y0b, sem).wait()
        pltpu.async_copy(yo_hbm.at[s1b], y1b, sem).wait()

        @pl.loop(0, ch)
        def _tok(i):
            for j in range(0, d_out, _LANES):
                sl = pl.ds(j, _LANES)
                y0b[i, sl] = y0b[i, sl] + y1b[i, sl]

        pltpu.sync_copy(y0b, out_hbm.at[pl.ds(t0, ch)])


def kernel(x, Wg, bg, W1, b1, W2, b2):
    n_tokens, d_in = x.shape
    ne = Wg.shape[-1]
    d_hid = W1.shape[-1]
    d_out = W2.shape[-1]
    bt = 256
    n_blocks = n_tokens // bt
    blk = 256
    nbmax = (2 * n_tokens) // blk + ne
    nslot = nbmax * blk
    tpw = n_tokens // _NW
    ch = 64
    ch2 = 32

    W1b = W1.astype(jnp.bfloat16)
    W2b = W2.astype(jnp.bfloat16)

    gate = pl.pallas_call(
        functools.partial(_gate_body, n_blocks=n_blocks, blk=blk,
                          nbmax=nbmax),
        grid=(n_blocks,),
        in_specs=[
            pl.BlockSpec((bt, d_in), lambda t: (t, 0)),
            pl.BlockSpec((d_in, ne), lambda t: (0, 0)),
            pl.BlockSpec((ne,), lambda t: (0,)),
        ],
        out_specs=[
            pl.BlockSpec((1, bt // 128, 128), lambda t: (t, 0, 0)),
            pl.BlockSpec((1, bt // 128, 128), lambda t: (t, 0, 0)),
            pl.BlockSpec((1, bt // 128, 128), lambda t: (t, 0, 0)),
            pl.BlockSpec((1, bt // 128, 128), lambda t: (t, 0, 0)),
            pl.BlockSpec((bt, 128), lambda t: (t, 0)),
            pl.BlockSpec((bt, 128), lambda t: (t, 0)),
            pl.BlockSpec((1, 128), lambda t: (0, 0)),
            pl.BlockSpec((1, 128), lambda t: (0, 0)),
        ],
        out_shape=[
            jax.ShapeDtypeStruct((n_blocks, bt // 128, 128), jnp.int32),
            jax.ShapeDtypeStruct((n_blocks, bt // 128, 128), jnp.int32),
            jax.ShapeDtypeStruct((n_blocks, bt // 128, 128), jnp.int32),
            jax.ShapeDtypeStruct((n_blocks, bt // 128, 128), jnp.int32),
            jax.ShapeDtypeStruct((n_tokens, 128), jnp.float32),
            jax.ShapeDtypeStruct((n_tokens, 128), jnp.float32),
            jax.ShapeDtypeStruct((1, 128), jnp.int32),
            jax.ShapeDtypeStruct((1, 128), jnp.int32),
        ],
        scratch_shapes=[pltpu.VMEM((1, ne), jnp.float32)],
        compiler_params=pltpu.CompilerParams(
            dimension_semantics=("arbitrary",)),
    )(x, Wg, bg)
    e0, e1, pos0, pos1, w0rep, w1rep, bex, off = gate
    e0 = e0.reshape(n_tokens)
    e1 = e1.reshape(n_tokens)
    pos0 = pos0.reshape(n_tokens)
    pos1 = pos1.reshape(n_tokens)
    bex = bex.reshape(128)
    off = off.reshape(128)

    sc_cp = pltpu.CompilerParams()
    if "needs_layout_passes" in pltpu.CompilerParams.__dataclass_fields__:
        sc_cp = dataclasses.replace(sc_cp, needs_layout_passes=False)
    mesh = plsc.VectorSubcoreMesh(core_axis_name="c", subcore_axis_name="s")
    scat = functools.partial(
        pl.kernel,
        mesh=mesh,
        compiler_params=sc_cp,
        out_type=[
            jax.ShapeDtypeStruct((nslot, d_in), jnp.float32),
            jax.ShapeDtypeStruct((nslot, 128), jnp.float32),
            jax.ShapeDtypeStruct((n_tokens,), jnp.int32),
            jax.ShapeDtypeStruct((n_tokens,), jnp.int32),
        ],
        scratch_types=[
            pltpu.VMEM((ch, d_in), jnp.float32),
            pltpu.VMEM((ch, 128), jnp.float32),
            pltpu.VMEM((ch, 128), jnp.float32),
            pltpu.VMEM((ch,), jnp.int32),
            pltpu.VMEM((ch,), jnp.int32),
            pltpu.VMEM((ch,), jnp.int32),
            pltpu.VMEM((ch,), jnp.int32),
            pltpu.VMEM((ch,), jnp.int32),
            pltpu.VMEM((ch,), jnp.int32),
            pltpu.VMEM((_LANES,), jnp.int32),
        ],
    )(functools.partial(_scat_body, tpw=tpw, ch=ch))
    xg, wslot, slot0, slot1 = scat(x, e0, e1, pos0, pos1, off, w0rep, w1rep)

    grid_spec = pltpu.PrefetchScalarGridSpec(
        num_scalar_prefetch=1,
        grid=(nbmax,),
        in_specs=[
            pl.BlockSpec((blk, d_in), lambda b, sp: (b, 0)),
            pl.BlockSpec((1, d_in, d_hid), lambda b, sp: (sp[b], 0, 0)),
            pl.BlockSpec((1, 1, d_hid), lambda b, sp: (sp[b], 0, 0)),
            pl.BlockSpec((1, d_hid, d_out), lambda b, sp: (sp[b], 0, 0)),
            pl.BlockSpec((1, 1, d_out), lambda b, sp: (sp[b], 0, 0)),
            pl.BlockSpec((blk, 128), lambda b, sp: (b, 0)),
        ],
        out_specs=pl.BlockSpec((blk, d_out), lambda b, sp: (b, 0)),
    )
    yo = pl.pallas_call(
        _ffn_body,
        grid_spec=grid_spec,
        out_shape=jax.ShapeDtypeStruct((nslot, d_out), jnp.float32),
        compiler_params=pltpu.CompilerParams(
            dimension_semantics=("arbitrary",)),
    )(bex, xg, W1b, b1.reshape(ne, 1, d_hid), W2b,
      b2.reshape(ne, 1, d_out), wslot)

    gath = functools.partial(
        pl.kernel,
        mesh=mesh,
        out_type=jax.ShapeDtypeStruct((n_tokens, d_out), jnp.float32),
        scratch_types=[
            pltpu.VMEM((ch2, d_out), jnp.float32),
            pltpu.VMEM((ch2, d_out), jnp.float32),
            pltpu.VMEM((ch2,), jnp.int32),
            pltpu.VMEM((ch2,), jnp.int32),
            pltpu.SemaphoreType.DMA,
        ],
    )(functools.partial(_gath_body, tpw=tpw, ch=ch2, d_out=d_out))
    out = gath(yo, slot0, slot1)

    gate_probs = jax.nn.softmax(x @ Wg + bg, axis=-1)
    expert_usage = jnp.mean(gate_probs, axis=0)
    uniform = jnp.ones_like(expert_usage) / ne
    aux = jnp.sum(expert_usage * jnp.log(uniform)
                  - jnp.log(expert_usage) * uniform)
    return (out, aux)


# speedup vs baseline: 5.9677x; 1.1127x over previous
import dataclasses
import functools
import math

import jax
import jax.numpy as jnp
from jax import lax
from jax.experimental import pallas as pl
from jax.experimental.pallas import tpu as pltpu
from jax.experimental.pallas import tpu_sc as plsc

_NC, _NS, _LANES = 2, 16, 16
_NW = _NC * _NS


def _gelu_exact(h):
    return 0.5 * h * (1.0 + lax.erf(h * (1.0 / math.sqrt(2.0))))


def _first_occurrence_count(eq):
    e = eq.shape[-1]
    r = lax.broadcasted_iota(jnp.int32, (e, e), 0)
    c = lax.broadcasted_iota(jnp.int32, (e, e), 1)
    tri = (r <= c).astype(jnp.bfloat16)
    return lax.dot_general(eq.astype(jnp.bfloat16), tri,
                           (((1,), (0,)), ((), ())),
                           preferred_element_type=jnp.float32)


def _gate_body(x_ref, wg_ref, bg_ref,
               e0_ref, e1_ref, p0_ref, p1_ref, w0_ref, w1_ref,
               bex_ref, off_ref, cnt_ref, *, n_blocks, blk, nbmax):
    t = pl.program_id(0)
    bt = x_ref.shape[0]
    ne = wg_ref.shape[-1]

    @pl.when(t == 0)
    def _init():
        cnt_ref[...] = jnp.zeros_like(cnt_ref)

    x = x_ref[...]
    logits = lax.dot_general(
        x.astype(jnp.bfloat16), wg_ref[...].astype(jnp.bfloat16),
        (((1,), (0,)), ((), ())),
        preferred_element_type=jnp.float32) + bg_ref[...][None, :]
    m = jnp.max(logits, axis=-1, keepdims=True)
    ex = jnp.exp(logits - m)
    p = ex / jnp.sum(ex, axis=-1, keepdims=True)

    m1 = jnp.max(p, axis=-1, keepdims=True)
    eq1 = p == m1
    oh1 = eq1 & (_first_occurrence_count(eq1) == 1.0)
    pm = jnp.where(oh1, -jnp.inf, p)
    m2 = jnp.max(pm, axis=-1, keepdims=True)
    eq2 = pm == m2
    oh2 = eq2 & (_first_occurrence_count(eq2) == 1.0)
    oh1f = oh1.astype(jnp.float32)
    oh2f = oh2.astype(jnp.float32)
    denom = m1 + m2

    ei = lax.broadcasted_iota(jnp.int32, (1, ne), 1).astype(jnp.float32)
    e0 = jnp.sum(oh1f * ei, axis=-1)
    e1 = jnp.sum(oh2f * ei, axis=-1)

    r = lax.broadcasted_iota(jnp.int32, (bt, bt), 0)
    c = lax.broadcasted_iota(jnp.int32, (bt, bt), 1)
    ls = (r > c).astype(jnp.bfloat16)
    f01 = (oh1f + oh2f).astype(jnp.bfloat16)
    cbefore = lax.dot_general(ls, f01, (((1,), (0,)), ((), ())),
                              preferred_element_type=jnp.float32)
    base = cnt_ref[...]
    pos0 = jnp.sum(oh1f * (base + cbefore), axis=-1)
    pos1 = jnp.sum(oh2f * (base + cbefore + oh1f), axis=-1)
    cnt_ref[...] = base + jnp.sum(oh1f + oh2f, axis=0, keepdims=True)

    shape2 = e0_ref.shape
    e0_ref[...] = e0.astype(jnp.int32).reshape(shape2)
    e1_ref[...] = e1.astype(jnp.int32).reshape(shape2)
    p0_ref[...] = pos0.astype(jnp.int32).reshape(shape2)
    p1_ref[...] = pos1.astype(jnp.int32).reshape(shape2)
    w0_ref[...] = jnp.broadcast_to(m1 / denom, w0_ref.shape)
    w1_ref[...] = jnp.broadcast_to(m2 / denom, w1_ref.shape)

    @pl.when(t == n_blocks - 1)
    def _finish():
        cnt = cnt_ref[...]
        nb = jnp.floor((cnt + (blk - 1)) / blk)
        r8 = lax.broadcasted_iota(jnp.int32, (ne, ne), 0)
        c8 = lax.broadcasted_iota(jnp.int32, (ne, ne), 1)
        tri8 = (r8 < c8).astype(jnp.bfloat16)
        blkoff = lax.dot_general(nb.astype(jnp.bfloat16), tri8,
                                 (((1,), (0,)), ((), ())),
                                 preferred_element_type=jnp.float32)
        off = (blkoff * blk).astype(jnp.int32)
        off_ref[...] = jnp.concatenate(
            [off, jnp.zeros((1, 128 - ne), jnp.int32)], axis=1)
        li = lax.broadcasted_iota(jnp.int32, (1, 128), 1)
        acc = jnp.full((1, 128), -1, jnp.int32)
        for e in range(ne):
            acc = acc + (li >= blkoff[0, e].astype(jnp.int32)).astype(jnp.int32)
        bex_ref[...] = jnp.clip(acc, 0, ne - 1)


def _ffn_body(sp_ref, xg_ref, w1_ref, b1_ref, w2_ref, b2_ref, ws_ref,
              yo_ref):
    xb = xg_ref[...].astype(jnp.bfloat16)
    h = lax.dot_general(xb, w1_ref[0], (((1,), (0,)), ((), ())),
                        preferred_element_type=jnp.float32) + b1_ref[0]
    hb = _gelu_exact(h).astype(jnp.bfloat16)
    o = lax.dot_general(hb, w2_ref[0], (((1,), (0,)), ((), ())),
                        preferred_element_type=jnp.float32) + b2_ref[0]
    yo_ref[...] = o * ws_ref[...][:, 0:1]


def _scat_body(x_hbm, e0_hbm, e1_hbm, p0_hbm, p1_hbm, off_hbm,
               w0_hbm, w1_hbm,
               xg_hbm, ws_hbm, s0_hbm, s1_hbm,
               e0b, e1b, p0b, p1b, s0b, s1b, offb,
               xbuf0, xbuf1, w0buf0, w0buf1, w1buf0, w1buf1, sem, lsem,
               *, tpw):
    wid = lax.axis_index("s") * _NC + lax.axis_index("c")
    base = pl.multiple_of(wid * tpw, tpw)
    pltpu.sync_copy(off_hbm.at[pl.ds(0, _LANES)], offb)
    pltpu.sync_copy(e0_hbm.at[pl.ds(base, tpw)], e0b)
    pltpu.sync_copy(e1_hbm.at[pl.ds(base, tpw)], e1b)
    pltpu.sync_copy(p0_hbm.at[pl.ds(base, tpw)], p0b)
    pltpu.sync_copy(p1_hbm.at[pl.ds(base, tpw)], p1b)
    ch = s0b.shape[1]
    nch = tpw // ch
    for i in range(0, tpw, _LANES):
        sl = pl.ds(i, _LANES)
        s0b[i // ch, pl.ds(i % ch, _LANES)] = (
            p0b[sl] + plsc.load_gather(offb, [e0b[sl]]))
        s1b[i // ch, pl.ds(i % ch, _LANES)] = (
            p1b[sl] + plsc.load_gather(offb, [e1b[sl]]))
    cbase = pl.multiple_of(base // ch, nch)
    pltpu.sync_copy(s0b, s0_hbm.at[pl.ds(cbase, nch)])
    pltpu.sync_copy(s1b, s1_hbm.at[pl.ds(cbase, nch)])
    xb = [xbuf0, xbuf1]
    wb0 = [w0buf0, w0buf1]
    wb1 = [w1buf0, w1buf1]
    loads = [None, None]
    scats = [None, None]

    def start_load(j):
        p = j % 2
        sl = pl.ds(pl.multiple_of(base + j * ch, ch), ch)
        loads[p] = (pltpu.async_copy(x_hbm.at[sl], xb[p], lsem),
                    pltpu.async_copy(w0_hbm.at[sl], wb0[p], lsem),
                    pltpu.async_copy(w1_hbm.at[sl], wb1[p], lsem))

    start_load(0)
    for j in range(nch):
        p = j % 2
        for h in loads[p]:
            h.wait()
        if j + 1 < nch:
            q = (j + 1) % 2
            if scats[q] is not None:
                for h in scats[q]:
                    h.wait()
                scats[q] = None
            start_load(j + 1)
        scats[p] = (
            pltpu.async_copy(xb[p], xg_hbm.at[s0b.at[j]], sem),
            pltpu.async_copy(xb[p], xg_hbm.at[s1b.at[j]], sem),
            pltpu.async_copy(wb0[p], ws_hbm.at[s0b.at[j]], sem),
            pltpu.async_copy(wb1[p], ws_hbm.at[s1b.at[j]], sem),
        )
    for s in scats:
        if s is not None:
            for h in s:
                h.wait()


def _gath_body(yo_hbm, s0_hbm, s1_hbm, out_hbm,
               y0a, y1a, y0b_, y1b_, s0b, s1b, sem, osem,
               *, tpw, ch, d_out):
    wid = lax.axis_index("s") * _NC + lax.axis_index("c")
    base = pl.multiple_of(wid * tpw, tpw)
    pltpu.sync_copy(s0_hbm.at[pl.ds(base, tpw)], s0b)
    pltpu.sync_copy(s1_hbm.at[pl.ds(base, tpw)], s1b)
    nch = tpw // ch
    y0 = [y0a, y0b_]
    y1 = [y1a, y1b_]

    def start(j):
        sl = pl.ds(j * ch, ch)
        return (pltpu.async_copy(yo_hbm.at[s0b.at[sl]], y0[j % 2], sem),
                pltpu.async_copy(yo_hbm.at[s1b.at[sl]], y1[j % 2], sem))

    pend = start(0)
    prev_out = [None, None]
    for j in range(nch):
        nxt = None
        if j + 1 < nch:
            if prev_out[(j + 1) % 2] is not None:
                prev_out[(j + 1) % 2].wait()
                prev_out[(j + 1) % 2] = None
            nxt = start(j + 1)
        pend[0].wait()
        pend[1].wait()
        a, b = y0[j % 2], y1[j % 2]

        @pl.loop(0, ch)
        def _tok(i):
            for k in range(0, d_out, _LANES):
                sl = pl.ds(k, _LANES)
                a[i, sl] = a[i, sl] + b[i, sl]

        prev_out[j % 2] = pltpu.async_copy(
            a, out_hbm.at[pl.ds(pl.multiple_of(base + j * ch, ch), ch)],
            osem)
        pend = nxt
    for h in prev_out:
        if h is not None:
            h.wait()


def kernel(x, Wg, bg, W1, b1, W2, b2):
    n_tokens, d_in = x.shape
    ne = Wg.shape[-1]
    d_hid = W1.shape[-1]
    d_out = W2.shape[-1]
    bt = 256
    n_blocks = n_tokens // bt
    blk = 256
    nbmax = (2 * n_tokens) // blk + ne
    nslot = nbmax * blk
    tpw = n_tokens // _NW
    ch = 32
    ch2 = 32

    W1b = W1.astype(jnp.bfloat16)
    W2b = W2.astype(jnp.bfloat16)

    gate = pl.pallas_call(
        functools.partial(_gate_body, n_blocks=n_blocks, blk=blk,
                          nbmax=nbmax),
        grid=(n_blocks,),
        in_specs=[
            pl.BlockSpec((bt, d_in), lambda t: (t, 0)),
            pl.BlockSpec((d_in, ne), lambda t: (0, 0)),
            pl.BlockSpec((ne,), lambda t: (0,)),
        ],
        out_specs=[
            pl.BlockSpec((1, bt // 128, 128), lambda t: (t, 0, 0)),
            pl.BlockSpec((1, bt // 128, 128), lambda t: (t, 0, 0)),
            pl.BlockSpec((1, bt // 128, 128), lambda t: (t, 0, 0)),
            pl.BlockSpec((1, bt // 128, 128), lambda t: (t, 0, 0)),
            pl.BlockSpec((bt, 128), lambda t: (t, 0)),
            pl.BlockSpec((bt, 128), lambda t: (t, 0)),
            pl.BlockSpec((1, 128), lambda t: (0, 0)),
            pl.BlockSpec((1, 128), lambda t: (0, 0)),
        ],
        out_shape=[
            jax.ShapeDtypeStruct((n_blocks, bt // 128, 128), jnp.int32),
            jax.ShapeDtypeStruct((n_blocks, bt // 128, 128), jnp.int32),
            jax.ShapeDtypeStruct((n_blocks, bt // 128, 128), jnp.int32),
            jax.ShapeDtypeStruct((n_blocks, bt // 128, 128), jnp.int32),
            jax.ShapeDtypeStruct((n_tokens, 128), jnp.float32),
            jax.ShapeDtypeStruct((n_tokens, 128), jnp.float32),
            jax.ShapeDtypeStruct((1, 128), jnp.int32),
            jax.ShapeDtypeStruct((1, 128), jnp.int32),
        ],
        scratch_shapes=[pltpu.VMEM((1, ne), jnp.float32)],
        compiler_params=pltpu.CompilerParams(
            dimension_semantics=("arbitrary",)),
    )(x, Wg, bg)
    e0, e1, pos0, pos1, w0rep, w1rep, bex, off = gate
    e0 = e0.reshape(n_tokens)
    e1 = e1.reshape(n_tokens)
    pos0 = pos0.reshape(n_tokens)
    pos1 = pos1.reshape(n_tokens)
    bex = bex.reshape(128)
    off = off.reshape(128)

    sc_cp = pltpu.CompilerParams()
    if "needs_layout_passes" in pltpu.CompilerParams.__dataclass_fields__:
        sc_cp = dataclasses.replace(sc_cp, needs_layout_passes=False)
    mesh = plsc.VectorSubcoreMesh(core_axis_name="c", subcore_axis_name="s")
    scat = functools.partial(
        pl.kernel,
        mesh=mesh,
        compiler_params=sc_cp,
        out_type=[
            jax.ShapeDtypeStruct((nslot, d_in), jnp.float32),
            jax.ShapeDtypeStruct((nslot, 128), jnp.float32),
            jax.ShapeDtypeStruct((n_tokens // ch, ch), jnp.int32),
            jax.ShapeDtypeStruct((n_tokens // ch, ch), jnp.int32),
        ],
        scratch_types=[
            pltpu.VMEM((tpw,), jnp.int32),
            pltpu.VMEM((tpw,), jnp.int32),
            pltpu.VMEM((tpw,), jnp.int32),
            pltpu.VMEM((tpw,), jnp.int32),
            pltpu.VMEM((tpw // ch, ch), jnp.int32),
            pltpu.VMEM((tpw // ch, ch), jnp.int32),
            pltpu.VMEM((_LANES,), jnp.int32),
            pltpu.VMEM((ch, d_in), jnp.float32),
            pltpu.VMEM((ch, d_in), jnp.float32),
            pltpu.VMEM((ch, 128), jnp.float32),
            pltpu.VMEM((ch, 128), jnp.float32),
            pltpu.VMEM((ch, 128), jnp.float32),
            pltpu.VMEM((ch, 128), jnp.float32),
            pltpu.SemaphoreType.DMA,
            pltpu.SemaphoreType.DMA,
        ],
    )(functools.partial(_scat_body, tpw=tpw))
    xg, wslot, slot0, slot1 = scat(x, e0, e1, pos0, pos1, off, w0rep, w1rep)
    slot0 = slot0.reshape(n_tokens)
    slot1 = slot1.reshape(n_tokens)

    grid_spec = pltpu.PrefetchScalarGridSpec(
        num_scalar_prefetch=1,
        grid=(nbmax,),
        in_specs=[
            pl.BlockSpec((blk, d_in), lambda b, sp: (b, 0)),
            pl.BlockSpec((1, d_in, d_hid), lambda b, sp: (sp[b], 0, 0)),
            pl.BlockSpec((1, 1, d_hid), lambda b, sp: (sp[b], 0, 0)),
            pl.BlockSpec((1, d_hid, d_out), lambda b, sp: (sp[b], 0, 0)),
            pl.BlockSpec((1, 1, d_out), lambda b, sp: (sp[b], 0, 0)),
            pl.BlockSpec((blk, 128), lambda b, sp: (b, 0)),
        ],
        out_specs=pl.BlockSpec((blk, d_out), lambda b, sp: (b, 0)),
    )
    yo = pl.pallas_call(
        _ffn_body,
        grid_spec=grid_spec,
        out_shape=jax.ShapeDtypeStruct((nslot, d_out), jnp.float32),
        compiler_params=pltpu.CompilerParams(
            dimension_semantics=("arbitrary",)),
    )(bex, xg, W1b, b1.reshape(ne, 1, d_hid), W2b,
      b2.reshape(ne, 1, d_out), wslot)

    gath = functools.partial(
        pl.kernel,
        mesh=mesh,
        out_type=jax.ShapeDtypeStruct((n_tokens, d_out), jnp.float32),
        scratch_types=[
            pltpu.VMEM((ch2, d_out), jnp.float32),
            pltpu.VMEM((ch2, d_out), jnp.float32),
            pltpu.VMEM((ch2, d_out), jnp.float32),
            pltpu.VMEM((ch2, d_out), jnp.float32),
            pltpu.VMEM((tpw,), jnp.int32),
            pltpu.VMEM((tpw,), jnp.int32),
            pltpu.SemaphoreType.DMA,
            pltpu.SemaphoreType.DMA,
        ],
    )(functools.partial(_gath_body, tpw=tpw, ch=ch2, d_out=d_out))
    out = gath(yo, slot0, slot1)

    gate_probs = jax.nn.softmax(x @ Wg + bg, axis=-1)
    expert_usage = jnp.mean(gate_probs, axis=0)
    uniform = jnp.ones_like(expert_usage) / ne
    aux = jnp.sum(expert_usage * jnp.log(uniform)
                  - jnp.log(expert_usage) * uniform)
    return (out, aux)


# speedup vs baseline: 6.3769x; 1.0686x over previous
import dataclasses
import functools
import math

import jax
import jax.numpy as jnp
from jax import lax
from jax.experimental import pallas as pl
from jax.experimental.pallas import tpu as pltpu
from jax.experimental.pallas import tpu_sc as plsc

_NC, _NS, _LANES = 2, 16, 16
_NW = _NC * _NS


def _gelu_exact(h):
    return 0.5 * h * (1.0 + lax.erf(h * (1.0 / math.sqrt(2.0))))


def _first_occurrence_count(eq):
    e = eq.shape[-1]
    r = lax.broadcasted_iota(jnp.int32, (e, e), 0)
    c = lax.broadcasted_iota(jnp.int32, (e, e), 1)
    tri = (r <= c).astype(jnp.bfloat16)
    return lax.dot_general(eq.astype(jnp.bfloat16), tri,
                           (((1,), (0,)), ((), ())),
                           preferred_element_type=jnp.float32)


def _gate_body(x_ref, wg_ref, bg_ref,
               e0_ref, e1_ref, p0_ref, p1_ref, w0_ref, w1_ref,
               bex_ref, off_ref, cnt_ref, *, n_blocks, blk, nbmax):
    t = pl.program_id(0)
    bt = x_ref.shape[0]
    ne = wg_ref.shape[-1]

    @pl.when(t == 0)
    def _init():
        cnt_ref[...] = jnp.zeros_like(cnt_ref)

    x = x_ref[...]
    logits = lax.dot_general(
        x.astype(jnp.bfloat16), wg_ref[...].astype(jnp.bfloat16),
        (((1,), (0,)), ((), ())),
        preferred_element_type=jnp.float32) + bg_ref[...][None, :]

    m1 = jnp.max(logits, axis=-1, keepdims=True)
    eq1 = logits == m1
    oh1 = eq1 & (_first_occurrence_count(eq1) == 1.0)
    pm = jnp.where(oh1, -jnp.inf, logits)
    m2 = jnp.max(pm, axis=-1, keepdims=True)
    eq2 = pm == m2
    oh2 = eq2 & (_first_occurrence_count(eq2) == 1.0)
    oh1f = oh1.astype(jnp.float32)
    oh2f = oh2.astype(jnp.float32)
    w0 = 1.0 / (1.0 + jnp.exp(m2 - m1))
    w1 = 1.0 / (1.0 + jnp.exp(m1 - m2))

    ei = lax.broadcasted_iota(jnp.int32, (1, ne), 1).astype(jnp.float32)
    e0 = jnp.sum(oh1f * ei, axis=-1)
    e1 = jnp.sum(oh2f * ei, axis=-1)

    r = lax.broadcasted_iota(jnp.int32, (bt, bt), 0)
    c = lax.broadcasted_iota(jnp.int32, (bt, bt), 1)
    ls = (r > c).astype(jnp.bfloat16)
    f01 = (oh1f + oh2f).astype(jnp.bfloat16)
    cbefore = lax.dot_general(ls, f01, (((1,), (0,)), ((), ())),
                              preferred_element_type=jnp.float32)
    base = cnt_ref[...]
    pos0 = jnp.sum(oh1f * (base + cbefore), axis=-1)
    pos1 = jnp.sum(oh2f * (base + cbefore + oh1f), axis=-1)
    cnt_ref[...] = base + jnp.sum(oh1f + oh2f, axis=0, keepdims=True)

    shape2 = e0_ref.shape
    e0_ref[...] = e0.astype(jnp.int32).reshape(shape2)
    e1_ref[...] = e1.astype(jnp.int32).reshape(shape2)
    p0_ref[...] = pos0.astype(jnp.int32).reshape(shape2)
    p1_ref[...] = pos1.astype(jnp.int32).reshape(shape2)
    w0_ref[...] = jnp.broadcast_to(w0, w0_ref.shape)
    w1_ref[...] = jnp.broadcast_to(w1, w1_ref.shape)

    @pl.when(t == n_blocks - 1)
    def _finish():
        cnt = cnt_ref[...]
        nb = jnp.floor((cnt + (blk - 1)) / blk)
        r8 = lax.broadcasted_iota(jnp.int32, (ne, ne), 0)
        c8 = lax.broadcasted_iota(jnp.int32, (ne, ne), 1)
        tri8 = (r8 < c8).astype(jnp.bfloat16)
        blkoff = lax.dot_general(nb.astype(jnp.bfloat16), tri8,
                                 (((1,), (0,)), ((), ())),
                                 preferred_element_type=jnp.float32)
        off = (blkoff * blk).astype(jnp.int32)
        off_ref[...] = jnp.concatenate(
            [off, jnp.zeros((1, 128 - ne), jnp.int32)], axis=1)
        li = lax.broadcasted_iota(jnp.int32, (1, 128), 1)
        acc = jnp.full((1, 128), -1, jnp.int32)
        for e in range(ne):
            acc = acc + (li >= blkoff[0, e].astype(jnp.int32)).astype(jnp.int32)
        bex_ref[...] = jnp.clip(acc, 0, ne - 1)


def _ffn_body(sp_ref, xg_ref, w1_ref, b1_ref, w2_ref, b2_ref, ws_ref,
              yo_ref):
    xb = xg_ref[...].astype(jnp.bfloat16)
    h = lax.dot_general(xb, w1_ref[0], (((1,), (0,)), ((), ())),
                        preferred_element_type=jnp.float32) + b1_ref[0]
    hb = _gelu_exact(h).astype(jnp.bfloat16)
    o = lax.dot_general(hb, w2_ref[0], (((1,), (0,)), ((), ())),
                        preferred_element_type=jnp.float32) + b2_ref[0]
    yo_ref[...] = o * ws_ref[...][:, 0:1]


def _scat_body(x_hbm, e0_hbm, e1_hbm, p0_hbm, p1_hbm, off_hbm,
               w0_hbm, w1_hbm,
               xg_hbm, ws_hbm, s0_hbm, s1_hbm,
               e0b, e1b, p0b, p1b, s0b, s1b, offb,
               xbuf0, xbuf1, w0buf0, w0buf1, w1buf0, w1buf1, sem, lsem,
               *, tpw):
    wid = lax.axis_index("s") * _NC + lax.axis_index("c")
    base = pl.multiple_of(wid * tpw, tpw)
    pltpu.sync_copy(off_hbm.at[pl.ds(0, _LANES)], offb)
    pltpu.sync_copy(e0_hbm.at[pl.ds(base, tpw)], e0b)
    pltpu.sync_copy(e1_hbm.at[pl.ds(base, tpw)], e1b)
    pltpu.sync_copy(p0_hbm.at[pl.ds(base, tpw)], p0b)
    pltpu.sync_copy(p1_hbm.at[pl.ds(base, tpw)], p1b)
    ch = s0b.shape[1]
    nch = tpw // ch
    for i in range(0, tpw, _LANES):
        sl = pl.ds(i, _LANES)
        s0b[i // ch, pl.ds(i % ch, _LANES)] = (
            p0b[sl] + plsc.load_gather(offb, [e0b[sl]]))
        s1b[i // ch, pl.ds(i % ch, _LANES)] = (
            p1b[sl] + plsc.load_gather(offb, [e1b[sl]]))
    cbase = pl.multiple_of(base // ch, nch)
    pltpu.sync_copy(s0b, s0_hbm.at[pl.ds(cbase, nch)])
    pltpu.sync_copy(s1b, s1_hbm.at[pl.ds(cbase, nch)])
    xb = [xbuf0, xbuf1]
    wb0 = [w0buf0, w0buf1]
    wb1 = [w1buf0, w1buf1]
    loads = [None, None]
    scats = [None, None]

    def start_load(j):
        p = j % 2
        sl = pl.ds(pl.multiple_of(base + j * ch, ch), ch)
        loads[p] = (pltpu.async_copy(x_hbm.at[sl], xb[p], lsem),
                    pltpu.async_copy(w0_hbm.at[sl], wb0[p], lsem),
                    pltpu.async_copy(w1_hbm.at[sl], wb1[p], lsem))

    start_load(0)
    for j in range(nch):
        p = j % 2
        for h in loads[p]:
            h.wait()
        if j + 1 < nch:
            q = (j + 1) % 2
            if scats[q] is not None:
                for h in scats[q]:
                    h.wait()
                scats[q] = None
            start_load(j + 1)
        scats[p] = (
            pltpu.async_copy(xb[p], xg_hbm.at[s0b.at[j]], sem),
            pltpu.async_copy(xb[p], xg_hbm.at[s1b.at[j]], sem),
            pltpu.async_copy(wb0[p], ws_hbm.at[s0b.at[j]], sem),
            pltpu.async_copy(wb1[p], ws_hbm.at[s1b.at[j]], sem),
        )
    for s in scats:
        if s is not None:
            for h in s:
                h.wait()


def _gath_body(yo_hbm, s0_hbm, s1_hbm, out_hbm,
               y0a, y1a, y0b_, y1b_, s0b, s1b, sem, osem,
               *, tpw, ch, d_out):
    wid = lax.axis_index("s") * _NC + lax.axis_index("c")
    base = pl.multiple_of(wid * tpw, tpw)
    pltpu.sync_copy(s0_hbm.at[pl.ds(base, tpw)], s0b)
    pltpu.sync_copy(s1_hbm.at[pl.ds(base, tpw)], s1b)
    nch = tpw // ch
    y0 = [y0a, y0b_]
    y1 = [y1a, y1b_]

    def start(j):
        sl = pl.ds(j * ch, ch)
        return (pltpu.async_copy(yo_hbm.at[s0b.at[sl]], y0[j % 2], sem),
                pltpu.async_copy(yo_hbm.at[s1b.at[sl]], y1[j % 2], sem))

    pend = start(0)
    prev_out = [None, None]
    for j in range(nch):
        nxt = None
        if j + 1 < nch:
            if prev_out[(j + 1) % 2] is not None:
                prev_out[(j + 1) % 2].wait()
                prev_out[(j + 1) % 2] = None
            nxt = start(j + 1)
        pend[0].wait()
        pend[1].wait()
        a, b = y0[j % 2], y1[j % 2]

        @pl.loop(0, ch)
        def _tok(i):
            for k in range(0, d_out, _LANES):
                sl = pl.ds(k, _LANES)
                a[i, sl] = a[i, sl] + b[i, sl]

        prev_out[j % 2] = pltpu.async_copy(
            a, out_hbm.at[pl.ds(pl.multiple_of(base + j * ch, ch), ch)],
            osem)
        pend = nxt
    for h in prev_out:
        if h is not None:
            h.wait()


def kernel(x, Wg, bg, W1, b1, W2, b2):
    n_tokens, d_in = x.shape
    ne = Wg.shape[-1]
    d_hid = W1.shape[-1]
    d_out = W2.shape[-1]
    bt = 256
    n_blocks = n_tokens // bt
    blk = 512
    nbmax = (2 * n_tokens) // blk + ne
    nslot = nbmax * blk
    tpw = n_tokens // _NW
    ch = 32
    ch2 = 32

    W1b = W1.astype(jnp.bfloat16)
    W2b = W2.astype(jnp.bfloat16)

    gate = pl.pallas_call(
        functools.partial(_gate_body, n_blocks=n_blocks, blk=blk,
                          nbmax=nbmax),
        grid=(n_blocks,),
        in_specs=[
            pl.BlockSpec((bt, d_in), lambda t: (t, 0)),
            pl.BlockSpec((d_in, ne), lambda t: (0, 0)),
            pl.BlockSpec((ne,), lambda t: (0,)),
        ],
        out_specs=[
            pl.BlockSpec((1, bt // 128, 128), lambda t: (t, 0, 0)),
            pl.BlockSpec((1, bt // 128, 128), lambda t: (t, 0, 0)),
            pl.BlockSpec((1, bt // 128, 128), lambda t: (t, 0, 0)),
            pl.BlockSpec((1, bt // 128, 128), lambda t: (t, 0, 0)),
            pl.BlockSpec((bt, 128), lambda t: (t, 0)),
            pl.BlockSpec((bt, 128), lambda t: (t, 0)),
            pl.BlockSpec((1, 128), lambda t: (0, 0)),
            pl.BlockSpec((1, 128), lambda t: (0, 0)),
        ],
        out_shape=[
            jax.ShapeDtypeStruct((n_blocks, bt // 128, 128), jnp.int32),
            jax.ShapeDtypeStruct((n_blocks, bt // 128, 128), jnp.int32),
            jax.ShapeDtypeStruct((n_blocks, bt // 128, 128), jnp.int32),
            jax.ShapeDtypeStruct((n_blocks, bt // 128, 128), jnp.int32),
            jax.ShapeDtypeStruct((n_tokens, 128), jnp.float32),
            jax.ShapeDtypeStruct((n_tokens, 128), jnp.float32),
            jax.ShapeDtypeStruct((1, 128), jnp.int32),
            jax.ShapeDtypeStruct((1, 128), jnp.int32),
        ],
        scratch_shapes=[pltpu.VMEM((1, ne), jnp.float32)],
        compiler_params=pltpu.CompilerParams(
            dimension_semantics=("arbitrary",)),
    )(x, Wg, bg)
    e0, e1, pos0, pos1, w0rep, w1rep, bex, off = gate
    e0 = e0.reshape(n_tokens)
    e1 = e1.reshape(n_tokens)
    pos0 = pos0.reshape(n_tokens)
    pos1 = pos1.reshape(n_tokens)
    bex = bex.reshape(128)
    off = off.reshape(128)

    sc_cp = pltpu.CompilerParams()
    if "needs_layout_passes" in pltpu.CompilerParams.__dataclass_fields__:
        sc_cp = dataclasses.replace(sc_cp, needs_layout_passes=False)
    mesh = plsc.VectorSubcoreMesh(core_axis_name="c", subcore_axis_name="s")
    scat = functools.partial(
        pl.kernel,
        mesh=mesh,
        compiler_params=sc_cp,
        out_type=[
            jax.ShapeDtypeStruct((nslot, d_in), jnp.float32),
            jax.ShapeDtypeStruct((nslot, 128), jnp.float32),
            jax.ShapeDtypeStruct((n_tokens // ch, ch), jnp.int32),
            jax.ShapeDtypeStruct((n_tokens // ch, ch), jnp.int32),
        ],
        scratch_types=[
            pltpu.VMEM((tpw,), jnp.int32),
            pltpu.VMEM((tpw,), jnp.int32),
            pltpu.VMEM((tpw,), jnp.int32),
            pltpu.VMEM((tpw,), jnp.int32),
            pltpu.VMEM((tpw // ch, ch), jnp.int32),
            pltpu.VMEM((tpw // ch, ch), jnp.int32),
            pltpu.VMEM((_LANES,), jnp.int32),
            pltpu.VMEM((ch, d_in), jnp.float32),
            pltpu.VMEM((ch, d_in), jnp.float32),
            pltpu.VMEM((ch, 128), jnp.float32),
            pltpu.VMEM((ch, 128), jnp.float32),
            pltpu.VMEM((ch, 128), jnp.float32),
            pltpu.VMEM((ch, 128), jnp.float32),
            pltpu.SemaphoreType.DMA,
            pltpu.SemaphoreType.DMA,
        ],
    )(functools.partial(_scat_body, tpw=tpw))
    xg, wslot, slot0, slot1 = scat(x, e0, e1, pos0, pos1, off, w0rep, w1rep)
    slot0 = slot0.reshape(n_tokens)
    slot1 = slot1.reshape(n_tokens)

    grid_spec = pltpu.PrefetchScalarGridSpec(
        num_scalar_prefetch=1,
        grid=(nbmax,),
        in_specs=[
            pl.BlockSpec((blk, d_in), lambda b, sp: (b, 0)),
            pl.BlockSpec((1, d_in, d_hid), lambda b, sp: (sp[b], 0, 0)),
            pl.BlockSpec((1, 1, d_hid), lambda b, sp: (sp[b], 0, 0)),
            pl.BlockSpec((1, d_hid, d_out), lambda b, sp: (sp[b], 0, 0)),
            pl.BlockSpec((1, 1, d_out), lambda b, sp: (sp[b], 0, 0)),
            pl.BlockSpec((blk, 128), lambda b, sp: (b, 0)),
        ],
        out_specs=pl.BlockSpec((blk, d_out), lambda b, sp: (b, 0)),
    )
    yo = pl.pallas_call(
        _ffn_body,
        grid_spec=grid_spec,
        out_shape=jax.ShapeDtypeStruct((nslot, d_out), jnp.float32),
        compiler_params=pltpu.CompilerParams(
            dimension_semantics=("arbitrary",)),
    )(bex, xg, W1b, b1.reshape(ne, 1, d_hid), W2b,
      b2.reshape(ne, 1, d_out), wslot)

    gath = functools.partial(
        pl.kernel,
        mesh=mesh,
        out_type=jax.ShapeDtypeStruct((n_tokens, d_out), jnp.float32),
        scratch_types=[
            pltpu.VMEM((ch2, d_out), jnp.float32),
            pltpu.VMEM((ch2, d_out), jnp.float32),
            pltpu.VMEM((ch2, d_out), jnp.float32),
            pltpu.VMEM((ch2, d_out), jnp.float32),
            pltpu.VMEM((tpw,), jnp.int32),
            pltpu.VMEM((tpw,), jnp.int32),
            pltpu.SemaphoreType.DMA,
            pltpu.SemaphoreType.DMA,
        ],
    )(functools.partial(_gath_body, tpw=tpw, ch=ch2, d_out=d_out))
    out = gath(yo, slot0, slot1)

    gate_probs = jax.nn.softmax(x @ Wg + bg, axis=-1)
    expert_usage = jnp.mean(gate_probs, axis=0)
    uniform = jnp.ones_like(expert_usage) / ne
    aux = jnp.sum(expert_usage * jnp.log(uniform)
                  - jnp.log(expert_usage) * uniform)
    return (out, aux)
